# baseline (speedup 1.0000x reference)
import functools

import jax
import jax.numpy as jnp
from jax import lax
from jax.experimental import pallas as pl
from jax.experimental.pallas import tpu as pltpu
from jax.experimental.pallas import tpu_sc as plsc

NUM_CORES = 2
NUM_SUBCORES = 16
NW = NUM_CORES * NUM_SUBCORES
CHUNK = 16
NBUF = 7


@functools.lru_cache(maxsize=None)
def _make_sc_gather(b: int, s: int, d: int):
    n_rows = b * s
    n_per_w = n_rows // NW
    n_chunks = n_per_w // CHUNK
    mesh = plsc.VectorSubcoreMesh(core_axis_name="c", subcore_axis_name="s")

    @functools.partial(
        pl.kernel,
        mesh=mesh,
        out_type=jax.ShapeDtypeStruct((n_rows, d), jnp.float32),
        scratch_types=[
            pltpu.VMEM((n_per_w,), jnp.int32),
            pltpu.VMEM((NBUF, CHUNK, d), jnp.float32),
            pltpu.SemaphoreType.DMA((NBUF,)),
            pltpu.SemaphoreType.DMA((NBUF,)),
            pltpu.SemaphoreType.DMA,
        ],
    )
    def k(idx_hbm, table_hbm, out_hbm, idx_v, rows_v, gsem, wsem, isem):
        wid = lax.axis_index("s") * NUM_CORES + lax.axis_index("c")
        base = wid * n_per_w
        head = 128
        row, col = base // s, base % s
        pltpu.sync_copy(idx_hbm.at[row, pl.ds(col, head)],
                        idx_v.at[pl.ds(0, head)])
        rest = pltpu.async_copy(
            idx_hbm.at[row, pl.ds(col + head, n_per_w - head)],
            idx_v.at[pl.ds(head, n_per_w - head)], isem)

        def gather(c, buf):
            return pltpu.make_async_copy(
                table_hbm.at[idx_v.at[pl.ds(c * CHUNK, CHUNK)]],
                rows_v.at[buf], gsem.at[buf])

        def write(c, buf):
            return pltpu.make_async_copy(
                rows_v.at[buf],
                out_hbm.at[pl.ds(base + c * CHUNK, CHUNK)], wsem.at[buf])

        def prime(c, _):
            gather(c, c).start()
            return 0

        lax.fori_loop(0, NBUF, prime, 0)
        rest.wait()
        for c in range(2):
            gather(c, c).wait()
            write(c, c).start()

        def body(c, _):
            buf = lax.rem(c, NBUF)
            gather(c, buf).wait()
            write(c, buf).start()
            pbuf = lax.rem(c - 2, NBUF)
            write(c - 2, pbuf).wait()
            gather(c - 2 + NBUF, pbuf).start()
            return 0

        lax.fori_loop(2, n_chunks - NBUF + 2, body, 0)

        def tail(c, _):
            buf = lax.rem(c, NBUF)
            gather(c, buf).wait()
            write(c, buf).start()
            return 0

        lax.fori_loop(n_chunks - NBUF + 2, n_chunks, tail, 0)

        def drain(c, _):
            write(c, lax.rem(c, NBUF)).wait()
            return 0

        lax.fori_loop(n_chunks - NBUF, n_chunks, drain, 0)

    return k


def kernel(input_ids, W):
    if input_ids.dtype != jnp.int32:
        input_ids = input_ids.astype(jnp.int32)
    b, s = input_ids.shape
    out = _make_sc_gather(b, s, W.shape[1])(input_ids, W)
    return out.reshape(b, s, W.shape[1])


# speedup vs baseline: 1.0043x; 1.0043x over previous
import functools

import jax
import jax.numpy as jnp
from jax import lax
from jax.experimental import pallas as pl
from jax.experimental.pallas import tpu as pltpu
from jax.experimental.pallas import tpu_sc as plsc

NUM_CORES = 2
NUM_SUBCORES = 16
NW = NUM_CORES * NUM_SUBCORES
CHUNK = 16
NBUF = 7


@functools.lru_cache(maxsize=None)
def _make_sc_gather(b: int, s: int, d: int):
    n_rows = b * s
    n_per_w = n_rows // NW
    n_chunks = n_per_w // CHUNK
    mesh = plsc.VectorSubcoreMesh(core_axis_name="c", subcore_axis_name="s")

    @functools.partial(
        pl.kernel,
        mesh=mesh,
        out_type=jax.ShapeDtypeStruct((n_rows, d), jnp.float32),
        scratch_types=[
            pltpu.VMEM((n_per_w,), jnp.int32),
            pltpu.VMEM((NBUF, CHUNK, d), jnp.float32),
            pltpu.SemaphoreType.DMA((NBUF,)),
            pltpu.SemaphoreType.DMA((NBUF,)),
            pltpu.SemaphoreType.DMA,
        ],
    )
    def k(idx_hbm, table_hbm, out_hbm, idx_v, rows_v, gsem, wsem, isem):
        wid = lax.axis_index("s") * NUM_CORES + lax.axis_index("c")
        base = wid * n_per_w
        head = 128
        row, col = base // s, base % s
        pltpu.sync_copy(idx_hbm.at[row, pl.ds(col, head)],
                        idx_v.at[pl.ds(0, head)])
        rest = pltpu.async_copy(
            idx_hbm.at[row, pl.ds(col + head, n_per_w - head)],
            idx_v.at[pl.ds(head, n_per_w - head)], isem)

        def gather(c, buf):
            return pltpu.make_async_copy(
                table_hbm.at[idx_v.at[pl.ds(c * CHUNK, CHUNK)]],
                rows_v.at[buf], gsem.at[buf])

        def write(c, buf):
            return pltpu.make_async_copy(
                rows_v.at[buf],
                out_hbm.at[pl.ds(base + c * CHUNK, CHUNK)], wsem.at[buf])

        def prime(c, _):
            gather(c, c).start()
            return 0

        lax.fori_loop(0, NBUF, prime, 0)
        rest.wait()
        gather(0, 0).wait()
        write(0, 0).start()

        def body(c, _):
            buf = lax.rem(c, NBUF)
            gather(c, buf).wait()
            write(c, buf).start()
            pbuf = lax.rem(c - 1, NBUF)
            write(c - 1, pbuf).wait()
            gather(c - 1 + NBUF, pbuf).start()
            return 0

        lax.fori_loop(1, n_chunks - NBUF + 1, body, 0)

        def tail(c, _):
            buf = lax.rem(c, NBUF)
            gather(c, buf).wait()
            write(c, buf).start()
            return 0

        lax.fori_loop(n_chunks - NBUF + 1, n_chunks, tail, 0)

        def drain(c, _):
            write(c, lax.rem(c, NBUF)).wait()
            return 0

        lax.fori_loop(n_chunks - NBUF, n_chunks, drain, 0)

    return k


def kernel(input_ids, W):
    if input_ids.dtype != jnp.int32:
        input_ids = input_ids.astype(jnp.int32)
    b, s = input_ids.shape
    out = _make_sc_gather(b, s, W.shape[1])(input_ids, W)
    return out.reshape(b, s, W.shape[1])
